# baseline (speedup 1.0000x reference)
import jax
import jax.numpy as jnp
from jax.experimental import pallas as pl
from jax.experimental.pallas import tpu as pltpu


_N_BLK = 2048


def _nnd_body(x_ref, yt_ref, d1_ref, d2_ref, rhs_ref):
    nb = pl.program_id(1)
    x = x_ref[0]
    n_blk = x.shape[0]
    bf16, f32 = jnp.bfloat16, jnp.float32

    @pl.when(nb == 0)
    def _build_rhs():
        yt = yt_ref[0]
        m = yt.shape[1]
        yb = yt.astype(bf16)
        y2 = jnp.sum(yt * yt, axis=0, keepdims=True)
        y2h = y2.astype(bf16)
        y2l = (y2 - y2h.astype(f32)).astype(bf16)
        ones_r = jnp.ones((1, m), bf16)
        rhs_ref[...] = jnp.concatenate(
            [yb, ones_r, ones_r, y2h, y2l, jnp.zeros((1, m), bf16)], axis=0)

    xm = ((-2.0) * x).astype(bf16)
    x2 = jnp.sum(x * x, axis=1, keepdims=True)
    x2h = x2.astype(bf16)
    x2l = (x2 - x2h.astype(f32)).astype(bf16)
    ones_c = jnp.ones((n_blk, 1), bf16)
    lhs = jnp.concatenate(
        [xm, x2h, x2l, ones_c, ones_c, jnp.zeros((n_blk, 1), bf16)], axis=1)

    d = jax.lax.dot_general(lhs, rhs_ref[...], (((1,), (0,)), ((), ())),
                            preferred_element_type=f32)

    d1_ref[0] = jnp.maximum(jnp.min(d, axis=1, keepdims=True), 0.0)

    cur = jnp.maximum(jnp.min(d, axis=0, keepdims=True), 0.0)

    @pl.when(nb == 0)
    def _init():
        d2_ref[0] = cur

    @pl.when(nb != 0)
    def _accum():
        d2_ref[0] = jnp.minimum(d2_ref[0], cur)


def kernel(input1, input2):
    B, N, _ = input1.shape
    M = input2.shape[1]
    yt = jnp.transpose(input2, (0, 2, 1))

    nb = N // _N_BLK
    out1, out2 = pl.pallas_call(
        _nnd_body,
        grid=(B, nb),
        in_specs=[
            pl.BlockSpec((1, _N_BLK, 3), lambda b, i: (b, i, 0)),
            pl.BlockSpec((1, 3, M), lambda b, i: (b, 0, 0)),
        ],
        out_specs=[
            pl.BlockSpec((1, _N_BLK, 1), lambda b, i: (b, i, 0)),
            pl.BlockSpec((1, 1, M), lambda b, i: (b, 0, 0)),
        ],
        out_shape=[
            jax.ShapeDtypeStruct((B, N, 1), jnp.float32),
            jax.ShapeDtypeStruct((B, 1, M), jnp.float32),
        ],
        scratch_shapes=[pltpu.VMEM((8, M), jnp.bfloat16)],
    )(input1, yt)
    return out1.reshape(B, N), out2.reshape(B, M)


# speedup vs baseline: 1.0119x; 1.0119x over previous
import jax
import jax.numpy as jnp
from jax.experimental import pallas as pl


_N_BLK = 2048


def _nnd_body(x_ref, yt_ref, d1_ref, d2_ref):
    nb = pl.program_id(1)
    x = x_ref[0]
    yt = yt_ref[0]

    n_blk = x.shape[0]
    m = yt.shape[1]
    bf16, f32 = jnp.bfloat16, jnp.float32

    xm = ((-2.0) * x).astype(bf16)
    yb = yt.astype(bf16)
    x2 = jnp.sum(x * x, axis=1, keepdims=True)
    y2 = jnp.sum(yt * yt, axis=0, keepdims=True)
    x2h = x2.astype(bf16)
    x2l = (x2 - x2h.astype(f32)).astype(bf16)
    y2h = y2.astype(bf16)
    y2l = (y2 - y2h.astype(f32)).astype(bf16)

    lhs = jnp.concatenate(
        [xm, x2h, x2l,
         jnp.ones((n_blk, 1), bf16), jnp.ones((n_blk, 1), bf16)], axis=1)

    mh = m // 2
    ones_h = jnp.ones((1, mh), bf16)
    dn = (((1,), (0,)), ((), ()))
    rhs_a = jnp.concatenate(
        [yb[:, :mh], ones_h, ones_h, y2h[:, :mh], y2l[:, :mh]], axis=0)
    da = jax.lax.dot_general(lhs, rhs_a, dn, preferred_element_type=f32)
    rhs_b = jnp.concatenate(
        [yb[:, mh:], ones_h, ones_h, y2h[:, mh:], y2l[:, mh:]], axis=0)
    db = jax.lax.dot_general(lhs, rhs_b, dn, preferred_element_type=f32)

    d1_ref[0] = jnp.maximum(
        jnp.minimum(jnp.min(da, axis=1, keepdims=True),
                    jnp.min(db, axis=1, keepdims=True)), 0.0)

    cur = jnp.maximum(
        jnp.concatenate([jnp.min(da, axis=0, keepdims=True),
                         jnp.min(db, axis=0, keepdims=True)], axis=1),
        0.0)

    @pl.when(nb == 0)
    def _init():
        d2_ref[0] = cur

    @pl.when(nb != 0)
    def _accum():
        d2_ref[0] = jnp.minimum(d2_ref[0], cur)


def kernel(input1, input2):
    B, N, _ = input1.shape
    M = input2.shape[1]
    yt = jnp.transpose(input2, (0, 2, 1))

    nb = N // _N_BLK
    out1, out2 = pl.pallas_call(
        _nnd_body,
        grid=(B, nb),
        in_specs=[
            pl.BlockSpec((1, _N_BLK, 3), lambda b, i: (b, i, 0)),
            pl.BlockSpec((1, 3, M), lambda b, i: (b, 0, 0)),
        ],
        out_specs=[
            pl.BlockSpec((1, _N_BLK, 1), lambda b, i: (b, i, 0)),
            pl.BlockSpec((1, 1, M), lambda b, i: (b, 0, 0)),
        ],
        out_shape=[
            jax.ShapeDtypeStruct((B, N, 1), jnp.float32),
            jax.ShapeDtypeStruct((B, 1, M), jnp.float32),
        ],
    )(input1, yt)
    return out1.reshape(B, N), out2.reshape(B, M)


# speedup vs baseline: 1.0122x; 1.0003x over previous
import jax
import jax.numpy as jnp
from jax.experimental import pallas as pl


_N_BLK = 2048


def _nnd_body(x_ref, yt_ref, d1_ref, d2_ref):
    nb = pl.program_id(1)
    x = x_ref[0]
    yt = yt_ref[0]

    n_blk = x.shape[0]
    m = yt.shape[1]
    bf16, f32 = jnp.bfloat16, jnp.float32

    xm = ((-2.0) * x).astype(bf16)
    yb = yt.astype(bf16)
    x2 = jnp.sum(x * x, axis=1, keepdims=True)
    y2 = jnp.sum(yt * yt, axis=0, keepdims=True)
    x2h = x2.astype(bf16)
    x2l = (x2 - x2h.astype(f32)).astype(bf16)
    y2h = y2.astype(bf16)
    y2l = (y2 - y2h.astype(f32)).astype(bf16)

    lhs = jnp.concatenate(
        [xm, x2h, x2l,
         jnp.ones((n_blk, 1), bf16), jnp.ones((n_blk, 1), bf16)], axis=1)
    rhs = jnp.concatenate(
        [yb, jnp.ones((1, m), bf16), jnp.ones((1, m), bf16),
         y2h, y2l], axis=0)

    d = jax.lax.dot_general(lhs, rhs, (((1,), (0,)), ((), ())),
                            preferred_element_type=f32)

    d1_ref[0] = jnp.maximum(jnp.min(d, axis=1, keepdims=True), 0.0)

    cur = jnp.maximum(jnp.min(d, axis=0, keepdims=True), 0.0)

    @pl.when(nb == 0)
    def _init():
        d2_ref[0] = cur

    @pl.when(nb != 0)
    def _accum():
        d2_ref[0] = jnp.minimum(d2_ref[0], cur)


def kernel(input1, input2):
    B, N, _ = input1.shape
    M = input2.shape[1]
    yt = jnp.transpose(input2, (0, 2, 1))

    nb = N // _N_BLK
    out1, out2 = pl.pallas_call(
        _nnd_body,
        grid=(B, nb),
        in_specs=[
            pl.BlockSpec((1, _N_BLK, 3), lambda b, i: (b, i, 0)),
            pl.BlockSpec((1, 3, M), lambda b, i: (b, 0, 0)),
        ],
        out_specs=[
            pl.BlockSpec((1, _N_BLK, 1), lambda b, i: (b, i, 0)),
            pl.BlockSpec((1, 1, M), lambda b, i: (b, 0, 0)),
        ],
        out_shape=[
            jax.ShapeDtypeStruct((B, N, 1), jnp.float32),
            jax.ShapeDtypeStruct((B, 1, M), jnp.float32),
        ],
    )(input1, yt)
    return out1.reshape(B, N), out2.reshape(B, M)
